# baseline (speedup 1.0000x reference)
import functools

import jax
import jax.numpy as jnp
from jax import lax
from jax.experimental import pallas as pl
from jax.experimental.pallas import tpu as pltpu
from jax.experimental.pallas import tpu_sc as plsc

_LENGTH = 1024
_VOCAB = 2049
_D = 128
_NC = 2
_NS = 16
_NW = _NC * _NS
_ROWS_PER_W = _LENGTH // _NW
_Q = 256
_NPH = _LENGTH // _Q
_WIN = _Q + _ROWS_PER_W

_mesh = plsc.VectorSubcoreMesh(core_axis_name="c", subcore_axis_name="s")


@functools.partial(
    pl.kernel,
    mesh=_mesh,
    out_type=jax.ShapeDtypeStruct((_LENGTH, _LENGTH, _D), jnp.float32),
    scratch_types=[
        pltpu.VMEM((2, _WIN, _D), jnp.float32),
        pltpu.SemaphoreType.DMA,
        pltpu.SemaphoreType.DMA,
    ],
)
def _sc_copy(table_hbm, out_hbm, buf, sem0, sem1):
    c = lax.axis_index("c")
    s = lax.axis_index("s")
    wid = s * _NC + c
    row0 = wid * _ROWS_PER_W
    base = 992 - _ROWS_PER_W * wid
    sems = (sem0, sem1)

    def _stage(p):
        pltpu.sync_copy(table_hbm.at[pl.ds(base + _Q * p, _WIN)], buf.at[p % 2])

    def _row(t, p):
        return pltpu.make_async_copy(
            buf.at[p % 2, pl.ds(_ROWS_PER_W - t, _Q)],
            out_hbm.at[row0 + t, pl.ds(_Q * p, _Q)],
            sems[p % 2],
        )

    def _fire(p):
        def body(t, cc):
            _row(t, p).start()
            return cc

        lax.fori_loop(0, _ROWS_PER_W, body, 0)

    def _drain(p):
        def body(t, cc):
            _row(t, p).wait()
            return cc

        lax.fori_loop(0, _ROWS_PER_W, body, 0)

    _stage(0)
    _fire(0)
    _stage(1)
    _fire(1)
    _drain(0)
    _stage(2)
    _fire(2)
    _drain(1)
    _stage(3)
    _fire(3)
    _drain(2)
    _drain(3)


def kernel(length, embedding_table):
    del length
    return _sc_copy(embedding_table)


# speedup vs baseline: 1.0143x; 1.0143x over previous
import functools

import jax
import jax.numpy as jnp
from jax import lax
from jax.experimental import pallas as pl
from jax.experimental.pallas import tpu as pltpu
from jax.experimental.pallas import tpu_sc as plsc

_LENGTH = 1024
_VOCAB = 2049
_D = 128
_NC = 2
_NS = 16
_NW = _NC * _NS
_ROWS_PER_W = _LENGTH // _NW
_WIN_A = 512 + _ROWS_PER_W
_WIN_B = 256 + _ROWS_PER_W

_mesh = plsc.VectorSubcoreMesh(core_axis_name="c", subcore_axis_name="s")


@functools.partial(
    pl.kernel,
    mesh=_mesh,
    out_type=jax.ShapeDtypeStruct((_LENGTH, _LENGTH, _D), jnp.float32),
    scratch_types=[
        pltpu.VMEM((_WIN_A, _D), jnp.float32),
        pltpu.VMEM((_WIN_B, _D), jnp.float32),
        pltpu.SemaphoreType.DMA,
        pltpu.SemaphoreType.DMA,
    ],
)
def _sc_copy(table_hbm, out_hbm, buf_a, buf_b, sem_a, sem_b):
    c = lax.axis_index("c")
    s = lax.axis_index("s")
    wid = s * _NC + c
    row0 = wid * _ROWS_PER_W
    base = 992 - _ROWS_PER_W * wid

    def _stage(buf, j0, win):
        pltpu.sync_copy(table_hbm.at[pl.ds(base + j0, win)], buf.at[pl.ds(0, win)])

    def _row(t, buf, j0, q, sem):
        return pltpu.make_async_copy(
            buf.at[pl.ds(_ROWS_PER_W - t, q)],
            out_hbm.at[row0 + t, pl.ds(j0, q)],
            sem,
        )

    def _fire(buf, j0, q, sem):
        def body(t, cc):
            _row(t, buf, j0, q, sem).start()
            return cc

        lax.fori_loop(0, _ROWS_PER_W, body, 0)

    def _drain(buf, j0, q, sem):
        def body(t, cc):
            _row(t, buf, j0, q, sem).wait()
            return cc

        lax.fori_loop(0, _ROWS_PER_W, body, 0)

    _stage(buf_a, 0, _WIN_A)
    _fire(buf_a, 0, 512, sem_a)
    _stage(buf_b, 512, _WIN_B)
    _fire(buf_b, 512, 256, sem_b)
    _drain(buf_a, 0, 512, sem_a)
    _stage(buf_a, 768, _WIN_B)
    _fire(buf_a, 768, 256, sem_a)
    _drain(buf_b, 512, 256, sem_b)
    _drain(buf_a, 768, 256, sem_a)


def kernel(length, embedding_table):
    del length
    return _sc_copy(embedding_table)


# speedup vs baseline: 1.0239x; 1.0094x over previous
import functools

import jax
import jax.numpy as jnp
from jax import lax
from jax.experimental import pallas as pl
from jax.experimental.pallas import tpu as pltpu
from jax.experimental.pallas import tpu_sc as plsc

_LENGTH = 1024
_VOCAB = 2049
_D = 128
_NC = 2
_NS = 16
_NW = _NC * _NS
_ROWS_PER_W = _LENGTH // _NW
_Q = 920
_TAIL = _LENGTH - _Q
_WIN = _Q + _ROWS_PER_W
_REG0 = 920
_REG = 1128

_mesh = plsc.VectorSubcoreMesh(core_axis_name="c", subcore_axis_name="s")


@functools.partial(
    pl.kernel,
    mesh=_mesh,
    out_type=jax.ShapeDtypeStruct((_LENGTH, _LENGTH, _D), jnp.float32),
    scratch_types=[
        pltpu.VMEM_SHARED((_REG, _D), jnp.float32),
        pltpu.VMEM((_WIN, _D), jnp.float32),
        pltpu.SemaphoreType.DMA,
        pltpu.SemaphoreType.DMA,
    ],
)
def _sc_copy(table_hbm, out_hbm, tail_sp, buf, sp_sem, st_sem):
    c = lax.axis_index("c")
    s = lax.axis_index("s")
    wid = s * _NC + c
    row0 = wid * _ROWS_PER_W
    base = 992 - _ROWS_PER_W * wid

    @pl.when(s == 0)
    def _stage_tail_region():
        pltpu.sync_copy(table_hbm.at[pl.ds(_REG0, _REG)], tail_sp)

    plsc.subcore_barrier()

    def _tail_row(t):
        i = row0 + t
        return pltpu.make_async_copy(
            tail_sp.at[pl.ds(_LENGTH + _Q - _REG0 - i, _TAIL)],
            out_hbm.at[i, pl.ds(_Q, _TAIL)],
            sp_sem,
        )

    def _tail_fire(t, cc):
        _tail_row(t).start()
        return cc

    lax.fori_loop(0, _ROWS_PER_W, _tail_fire, 0)

    pltpu.sync_copy(table_hbm.at[pl.ds(base, _WIN)], buf)

    def _bulk_row(t):
        return pltpu.make_async_copy(
            buf.at[pl.ds(_ROWS_PER_W - t, _Q)],
            out_hbm.at[row0 + t, pl.ds(0, _Q)],
            st_sem,
        )

    def _bulk_fire(t, cc):
        _bulk_row(t).start()
        return cc

    def _bulk_drain(t, cc):
        _bulk_row(t).wait()
        return cc

    def _tail_drain(t, cc):
        _tail_row(t).wait()
        return cc

    lax.fori_loop(0, _ROWS_PER_W, _bulk_fire, 0)
    lax.fori_loop(0, _ROWS_PER_W, _tail_drain, 0)
    lax.fori_loop(0, _ROWS_PER_W, _bulk_drain, 0)


def kernel(length, embedding_table):
    del length
    return _sc_copy(embedding_table)


# speedup vs baseline: 1.0336x; 1.0095x over previous
import functools

import jax
import jax.numpy as jnp
from jax import lax
from jax.experimental import pallas as pl
from jax.experimental.pallas import tpu as pltpu
from jax.experimental.pallas import tpu_sc as plsc

_LENGTH = 1024
_VOCAB = 2049
_D = 128
_NC = 2
_NS = 16
_NW = _NC * _NS
_ROWS_PER_W = _LENGTH // _NW
_WIN = 544

_mesh = plsc.VectorSubcoreMesh(core_axis_name="c", subcore_axis_name="s")


@functools.partial(
    pl.kernel,
    mesh=_mesh,
    out_type=jax.ShapeDtypeStruct((_LENGTH, _LENGTH, _D), jnp.float32),
    scratch_types=[
        pltpu.VMEM((_WIN, _D), jnp.float32),
        pltpu.SemaphoreType.DMA,
    ],
)
def _sc_copy(table_hbm, out_hbm, buf, sem):
    c = lax.axis_index("c")
    s = lax.axis_index("s")
    wid = s * _NC + c
    base = 992 - _ROWS_PER_W * wid

    def _phase(p, carry):
        start = base + 512 * p
        pltpu.sync_copy(table_hbm.at[pl.ds(start, _WIN)], buf)

        def _row(t, i):
            return pltpu.make_async_copy(
                buf.at[pl.ds(_ROWS_PER_W - t, 512)],
                out_hbm.at[i, pl.ds(512 * p, 512)],
                sem,
            )

        def _fire(t, cc):
            _row(t, wid * _ROWS_PER_W + t).start()
            return cc

        def _drain(t, cc):
            _row(t, wid * _ROWS_PER_W + t).wait()
            return cc

        lax.fori_loop(0, _ROWS_PER_W, _fire, 0)
        lax.fori_loop(0, _ROWS_PER_W, _drain, 0)
        return carry

    lax.fori_loop(0, 2, _phase, 0)


def kernel(length, embedding_table):
    del length
    return _sc_copy(embedding_table)
